# baseline (speedup 1.0000x reference)
import functools

import jax
import jax.numpy as jnp
from jax import lax
from jax.experimental import pallas as pl
from jax.experimental.pallas import tpu as pltpu
from jax.experimental.pallas import tpu_sc as plsc

B = 16384
D = 64
NC = 2
NS = 16
NW = NC * NS
BPW = B // NW
CHUNK = 128
NCHUNK = BPW // CHUNK


def _body(h_hbm, r_hbm, t_hbm, ent_hbm, rel_hbm, out_hbm,
          hi, ri, ti, hv, rv, tv, sem):
    wid = lax.axis_index("s") * NC + lax.axis_index("c")
    base = wid * BPW
    pltpu.sync_copy(h_hbm.at[pl.ds(wid * NCHUNK, NCHUNK)], hi)
    pltpu.sync_copy(r_hbm.at[pl.ds(wid * NCHUNK, NCHUNK)], ri)
    pltpu.sync_copy(t_hbm.at[pl.ds(wid * NCHUNK, NCHUNK)], ti)
    copies = []
    for c in range(NCHUNK):
        rows = pl.ds(c * CHUNK, CHUNK)
        copies.append(pltpu.async_copy(ent_hbm.at[hi.at[c]], hv.at[rows], sem))
        copies.append(pltpu.async_copy(rel_hbm.at[ri.at[c]], rv.at[rows], sem))
        copies.append(pltpu.async_copy(ent_hbm.at[ti.at[c]], tv.at[rows], sem))
    for cp in copies:
        cp.wait()

    def row(i, _):
        for j in range(D // 16):
            s = pl.ds(j * 16, 16)
            hv[i, s] = hv[i, s] + rv[i, s] - tv[i, s]
        return _

    lax.fori_loop(0, BPW, row, None)
    pltpu.sync_copy(hv, out_hbm.at[pl.ds(base, BPW)])


@jax.jit
def kernel(h, r, t, entity_table, relation_table):
    mesh = plsc.VectorSubcoreMesh(core_axis_name="c", subcore_axis_name="s")
    k = functools.partial(
        pl.kernel,
        mesh=mesh,
        compiler_params=pltpu.CompilerParams(use_tc_tiling_on_sc=False),
        out_type=jax.ShapeDtypeStruct((B, D), jnp.float32),
        scratch_types=[
            pltpu.VMEM((NCHUNK, CHUNK), jnp.int32),
            pltpu.VMEM((NCHUNK, CHUNK), jnp.int32),
            pltpu.VMEM((NCHUNK, CHUNK), jnp.int32),
            pltpu.VMEM((BPW, D), jnp.float32),
            pltpu.VMEM((BPW, D), jnp.float32),
            pltpu.VMEM((BPW, D), jnp.float32),
            pltpu.SemaphoreType.DMA,
        ],
    )(_body)
    h2 = h.reshape(B // CHUNK, CHUNK)
    r2 = r.reshape(B // CHUNK, CHUNK)
    t2 = t.reshape(B // CHUNK, CHUNK)
    return k(h2, r2, t2, entity_table, relation_table)
